# baseline (speedup 1.0000x reference)
import functools

import jax
import jax.numpy as jnp
from jax import lax
from jax.experimental import pallas as pl
from jax.experimental.pallas import tpu as pltpu
from jax.experimental.pallas import tpu_sc as plsc

N = 10000
NP = 10240
E = 320000
D = 128
G = 16
NC, NS, L = 2, 16, 16
NW = NC * NS
CH = 64
EP = 327680
EPW0 = 18432
EPW1 = 2048
HCW0 = EPW0 // CH
HCW1 = EPW1 // CH
GPC = CH // L
DUMMY_DST = N + 100
RT = NP // NS

_mesh = plsc.VectorSubcoreMesh(core_axis_name="c", subcore_axis_name="s")
_sc_params = pltpu.CompilerParams(needs_layout_passes=False)


@functools.partial(
    pl.kernel,
    out_type=[
        jax.ShapeDtypeStruct((NC, NP, D), jnp.float32),
        jax.ShapeDtypeStruct((NC, NP, D), jnp.float32),
    ],
    mesh=_mesh,
    compiler_params=_sc_params,
    scratch_types=[
        pltpu.VMEM_SHARED((NP, D), jnp.float32),
        pltpu.VMEM_SHARED((NP,), jnp.float32),
        pltpu.VMEM((2, NP), jnp.float32),
        pltpu.VMEM((CH,), jnp.int32),
        pltpu.VMEM((CH,), jnp.int32),
        pltpu.VMEM((CH,), jnp.int32),
        pltpu.VMEM((CH,), jnp.int32),
        pltpu.VMEM((CH,), jnp.int32),
        pltpu.VMEM((CH,), jnp.int32),
        pltpu.VMEM((CH,), jnp.int32),
        pltpu.VMEM((CH,), jnp.int32),
        pltpu.VMEM((CH, D), jnp.float32),
        pltpu.VMEM((CH, D), jnp.float32),
        pltpu.VMEM((CH,), jnp.float32),
        pltpu.VMEM((CH,), jnp.float32),
        pltpu.VMEM((RT,), jnp.float32),
        pltpu.SemaphoreType.DMA,
        pltpu.SemaphoreType.DMA,
        pltpu.SemaphoreType.DMA,
        pltpu.SemaphoreType.DMA,
        pltpu.SemaphoreType.DMA,
        pltpu.SemaphoreType.DMA,
        pltpu.SemaphoreType.DMA,
        pltpu.SemaphoreType.DMA,
    ],
)
def _gat_edges(h_hbm, asad_hbm, src_hbm, dst_hbm, num_out, den_out,
               acc_sh, den_sh, asad_v,
               sb0, sb1, sb2, sb3, db0, db1, db2, db3,
               rows0, rows1, wc0, wc1, sbuf,
               sg0, sg1, ss0, ss1, si0, si1, si2, si3):
    cid = lax.axis_index("c")
    sid = lax.axis_index("s")
    wbase = jnp.where(cid == 0, sid * EPW0, NS * EPW0 + sid * EPW1)
    hcw = jnp.where(cid == 0, HCW0, HCW1)
    zeros16 = jnp.zeros((L,), jnp.float32)
    sb = [sb0, sb1, sb2, sb3]
    db = [db0, db1, db2, db3]
    si = [si0, si1, si2, si3]
    rows = [rows0, rows1]
    wc = [wc0, wc1]
    sg = [sg0, sg1]
    ss = [ss0, ss1]
    bcast_dn = lax.GatherDimensionNumbers(
        offset_dims=(), collapsed_slice_dims=(0,), start_index_map=(0,))

    def _bcast(vec, i):
        return lax.gather(vec, jnp.full((L, 1), i, jnp.int32), bcast_dn, (1,),
                          mode=lax.GatherScatterMode.PROMISE_IN_BOUNDS)

    def idx_start(hc, k):
        base = wbase + hc * CH
        pltpu.async_copy(src_hbm.at[pl.ds(base, CH)], sb[k], si[k])
        pltpu.async_copy(dst_hbm.at[pl.ds(base, CH)], db[k], si[k])

    def idx_wait(k):
        pltpu.make_async_copy(src_hbm.at[pl.ds(0, CH)], sb[k], si[k]).wait()
        pltpu.make_async_copy(dst_hbm.at[pl.ds(0, CH)], db[k], si[k]).wait()

    def gather_start(p, k):
        pltpu.async_copy(h_hbm.at[sb[k]], rows[p], sg[p])

    def gather_wait(p, k):
        pltpu.make_async_copy(h_hbm.at[sb[k]], rows[p], sg[p]).wait()

    def scatter_start(p, k):
        pltpu.async_copy(rows[p], acc_sh.at[db[k]], ss[p], add=True)
        pltpu.async_copy(wc[p], den_sh.at[db[k]], ss[p], add=True)

    def scatter_wait(p, k):
        pltpu.make_async_copy(rows[p], acc_sh.at[db[k]], ss[p]).wait()
        pltpu.make_async_copy(wc[p], den_sh.at[db[k]], ss[p]).wait()

    pltpu.sync_copy(asad_hbm, asad_v)

    def _zrows(i, c):
        for j in range(D // L):
            rows0[i, pl.ds(j * L, L)] = zeros16
        return c
    lax.fori_loop(0, CH, _zrows, 0)

    def _zsbuf(i, c):
        sbuf[pl.ds(i * L, L)] = zeros16
        return c
    lax.fori_loop(0, RT // L, _zsbuf, 0)
    for b in range(RT // CH):
        pltpu.sync_copy(rows0, acc_sh.at[pl.ds(sid * RT + b * CH, CH)])
    pltpu.sync_copy(sbuf, den_sh.at[pl.ds(sid * RT, RT)])
    plsc.subcore_barrier()

    def compute(p, k):
        def _grp(g, cc):
            sv = sb[k][pl.ds(g * L, L)]
            dv = db[k][pl.ds(g * L, L)]
            a_s = plsc.load_gather(asad_v, [jnp.zeros((L,), jnp.int32), sv])
            a_d = plsc.load_gather(asad_v, [jnp.ones((L,), jnp.int32), dv])
            e = a_s + a_d
            e = jnp.maximum(e, e * 0.2)
            w = jnp.exp(e)
            wc[p][pl.ds(g * L, L)] = w
            for ei in range(L):
                wb = _bcast(w, ei)
                r = g * L + ei
                for j in range(D // L):
                    rows[p][r, pl.ds(j * L, L)] = \
                        rows[p][r, pl.ds(j * L, L)] * wb
            return cc
        lax.fori_loop(0, GPC, _grp, 0)

    idx_start(0, 0)
    idx_start(1, 1)
    idx_wait(0)
    gather_start(0, 0)

    def _quad(q, carry):
        for k4 in range(4):
            p = k4 & 1
            pn = p ^ 1
            hc = q * 4 + k4
            gather_wait(p, k4)

            @pl.when(hc >= 1)
            def _():
                scatter_wait(pn, (k4 + 3) % 4)

            @pl.when(hc + 1 < hcw)
            def _():
                idx_wait((k4 + 1) % 4)
                gather_start(pn, (k4 + 1) % 4)

            compute(p, k4)
            scatter_start(p, k4)

            @pl.when(hc + 2 < hcw)
            def _():
                idx_start(hc + 2, (k4 + 2) % 4)
        return carry
    lax.fori_loop(0, hcw // 4, _quad, 0)
    scatter_wait(1, 3)
    plsc.subcore_barrier()

    pltpu.sync_copy(den_sh.at[pl.ds(sid * RT, RT)], sbuf)
    for b in range(RT // CH):
        def _red(g, cc):
            dvals = sbuf[pl.ds(b * CH + g * L, L)]
            for ei in range(L):
                dsp = _bcast(dvals, ei)
                r = g * L + ei
                for j in range(D // L):
                    rows0[r, pl.ds(j * L, L)] = dsp
            return cc
        lax.fori_loop(0, GPC, _red, 0)
        pltpu.sync_copy(
            rows0, den_out.at[cid, pl.ds(sid * RT + b * CH, CH)])

    pltpu.sync_copy(acc_sh.at[pl.ds(sid * RT, RT)],
                    num_out.at[cid, pl.ds(sid * RT, RT)])


_GRID = 8
_BR = NP // _GRID


def _head_body(x_ref, W_ref, aa_ref, h_ref, asad_ref):
    h = jnp.dot(x_ref[...], W_ref[...], preferred_element_type=jnp.float32)
    h_ref[...] = h
    asad_ref[...] = lax.dot_general(
        aa_ref[...], h, (((1,), (1,)), ((), ())),
        preferred_element_type=jnp.float32)


_head = pl.pallas_call(
    _head_body,
    grid=(_GRID,),
    in_specs=[
        pl.BlockSpec((_BR, D), lambda i: (i, 0)),
        pl.BlockSpec((D, D), lambda i: (0, 0)),
        pl.BlockSpec((2, D), lambda i: (0, 0)),
    ],
    out_specs=[
        pl.BlockSpec((_BR, D), lambda i: (i, 0)),
        pl.BlockSpec((2, _BR), lambda i: (0, i)),
    ],
    out_shape=[
        jax.ShapeDtypeStruct((NP, D), jnp.float32),
        jax.ShapeDtypeStruct((2, NP), jnp.float32),
    ],
)


def _mid_body(num_ref, den_ref, b_ref, W_ref, aa_ref, h_ref, asad_ref):
    ntot = num_ref[0] + num_ref[1]
    dtot = den_ref[0] + den_ref[1]
    hact = jnp.maximum(ntot / (dtot + 1e-16) + b_ref[...], 0.0)
    h2 = jnp.dot(hact, W_ref[...], preferred_element_type=jnp.float32)
    h_ref[...] = h2
    asad_ref[...] = lax.dot_general(
        aa_ref[...], h2, (((1,), (1,)), ((), ())),
        preferred_element_type=jnp.float32)


_mid = pl.pallas_call(
    _mid_body,
    grid=(_GRID,),
    in_specs=[
        pl.BlockSpec((NC, _BR, D), lambda i: (0, i, 0)),
        pl.BlockSpec((NC, _BR, D), lambda i: (0, i, 0)),
        pl.BlockSpec((1, D), lambda i: (0, 0)),
        pl.BlockSpec((D, D), lambda i: (0, 0)),
        pl.BlockSpec((2, D), lambda i: (0, 0)),
    ],
    out_specs=[
        pl.BlockSpec((_BR, D), lambda i: (i, 0)),
        pl.BlockSpec((2, _BR), lambda i: (0, i)),
    ],
    out_shape=[
        jax.ShapeDtypeStruct((NP, D), jnp.float32),
        jax.ShapeDtypeStruct((2, NP), jnp.float32),
    ],
)


def _tail_body(num_ref, den_ref, b_ref, Wp_ref, bp_ref, batch_ref, out_ref,
               sums, cnts):
    i = pl.program_id(0)

    @pl.when(i == 0)
    def _():
        sums[...] = jnp.zeros((G, D), jnp.float32)
        cnts[...] = jnp.zeros((G, D), jnp.float32)

    ntot = num_ref[0] + num_ref[1]
    dtot = den_ref[0] + den_ref[1]
    hact = jnp.maximum(ntot / (dtot + 1e-16) + b_ref[...], 0.0)
    scores = jnp.dot(hact, Wp_ref[...],
                     preferred_element_type=jnp.float32) + bp_ref[...]
    gids = lax.broadcasted_iota(jnp.int32, (_BR, G), 1)
    oh = (batch_ref[...] == gids).astype(jnp.float32)
    sums[...] += lax.dot_general(
        oh, scores, (((0,), (0,)), ((), ())),
        preferred_element_type=jnp.float32)
    cnts[...] += lax.dot_general(
        oh, jnp.ones((_BR, D), jnp.float32), (((0,), (0,)), ((), ())),
        preferred_element_type=jnp.float32)

    @pl.when(i == _GRID - 1)
    def _():
        out_ref[...] = sums[...] / jnp.maximum(cnts[...], 1.0)


_tail = pl.pallas_call(
    _tail_body,
    grid=(_GRID,),
    in_specs=[
        pl.BlockSpec((NC, _BR, D), lambda i: (0, i, 0)),
        pl.BlockSpec((NC, _BR, D), lambda i: (0, i, 0)),
        pl.BlockSpec((1, D), lambda i: (0, 0)),
        pl.BlockSpec((D, D), lambda i: (0, 0)),
        pl.BlockSpec((1, D), lambda i: (0, 0)),
        pl.BlockSpec((_BR, 1), lambda i: (i, 0)),
    ],
    out_specs=pl.BlockSpec((G, D), lambda i: (0, 0)),
    out_shape=jax.ShapeDtypeStruct((G, D), jnp.float32),
    scratch_shapes=[
        pltpu.VMEM((G, D), jnp.float32),
        pltpu.VMEM((G, D), jnp.float32),
    ],
)


def kernel(x, edge_index, batch, W1, a1_src, a1_dst, b1, W2, a2_src, a2_dst,
           b2, Wp, bp):
    src = edge_index[0].astype(jnp.int32)
    dst = edge_index[1].astype(jnp.int32)
    src_p = jnp.concatenate([src, jnp.zeros((EP - E,), jnp.int32)])
    dst_p = jnp.concatenate([dst, jnp.full((EP - E,), DUMMY_DST, jnp.int32)])
    x_p = jnp.pad(x, ((0, NP - N), (0, 0)))
    batch_p = jnp.concatenate(
        [batch.astype(jnp.int32), jnp.full((NP - N,), G, jnp.int32)]
    ).reshape(NP, 1)
    aa1 = jnp.stack([a1_src, a1_dst])
    aa2 = jnp.stack([a2_src, a2_dst])

    h1, asad1 = _head(x_p, W1, aa1)
    num1, den1 = _gat_edges(h1, asad1, src_p, dst_p)
    h2, asad2 = _mid(num1, den1, b1.reshape(1, D), W2, aa2)
    num2, den2 = _gat_edges(h2, asad2, src_p, dst_p)
    return _tail(num2, den2, b2.reshape(1, D), Wp, bp.reshape(1, D), batch_p)


# speedup vs baseline: 1.0672x; 1.0672x over previous
import functools

import jax
import jax.numpy as jnp
from jax import lax
from jax.experimental import pallas as pl
from jax.experimental.pallas import tpu as pltpu
from jax.experimental.pallas import tpu_sc as plsc

N = 10000
NP = 10240
E = 320000
D = 128
G = 16
NC, NS, L = 2, 16, 16
NW = NC * NS
CH = 64
EP = 327680
EPW0 = 17408
EPW1 = 3072
HCW0 = EPW0 // CH
HCW1 = EPW1 // CH
GPC = CH // L
DUMMY_DST = N + 100
RT = NP // NS

_mesh = plsc.VectorSubcoreMesh(core_axis_name="c", subcore_axis_name="s")
_sc_params = pltpu.CompilerParams(needs_layout_passes=False)


@functools.partial(
    pl.kernel,
    out_type=[
        jax.ShapeDtypeStruct((NC, NP, D), jnp.float32),
        jax.ShapeDtypeStruct((NC, NP, D), jnp.float32),
    ],
    mesh=_mesh,
    compiler_params=_sc_params,
    scratch_types=[
        pltpu.VMEM_SHARED((NP, D), jnp.float32),
        pltpu.VMEM_SHARED((NP,), jnp.float32),
        pltpu.VMEM((2, NP), jnp.float32),
        pltpu.VMEM((CH,), jnp.int32),
        pltpu.VMEM((CH,), jnp.int32),
        pltpu.VMEM((CH,), jnp.int32),
        pltpu.VMEM((CH,), jnp.int32),
        pltpu.VMEM((CH,), jnp.int32),
        pltpu.VMEM((CH,), jnp.int32),
        pltpu.VMEM((CH,), jnp.int32),
        pltpu.VMEM((CH,), jnp.int32),
        pltpu.VMEM((CH, D), jnp.float32),
        pltpu.VMEM((CH, D), jnp.float32),
        pltpu.VMEM((CH,), jnp.float32),
        pltpu.VMEM((CH,), jnp.float32),
        pltpu.VMEM((RT,), jnp.float32),
        pltpu.SemaphoreType.DMA,
        pltpu.SemaphoreType.DMA,
        pltpu.SemaphoreType.DMA,
        pltpu.SemaphoreType.DMA,
        pltpu.SemaphoreType.DMA,
        pltpu.SemaphoreType.DMA,
        pltpu.SemaphoreType.DMA,
        pltpu.SemaphoreType.DMA,
    ],
)
def _gat_edges(h_hbm, asad_hbm, src_hbm, dst_hbm, num_out, den_out,
               acc_sh, den_sh, asad_v,
               sb0, sb1, sb2, sb3, db0, db1, db2, db3,
               rows0, rows1, wc0, wc1, sbuf,
               sg0, sg1, ss0, ss1, si0, si1, si2, si3):
    cid = lax.axis_index("c")
    sid = lax.axis_index("s")
    wbase = jnp.where(cid == 0, sid * EPW0, NS * EPW0 + sid * EPW1)
    hcw = jnp.where(cid == 0, HCW0, HCW1)
    zeros16 = jnp.zeros((L,), jnp.float32)
    sb = [sb0, sb1, sb2, sb3]
    db = [db0, db1, db2, db3]
    si = [si0, si1, si2, si3]
    rows = [rows0, rows1]
    wc = [wc0, wc1]
    sg = [sg0, sg1]
    ss = [ss0, ss1]
    bcast_dn = lax.GatherDimensionNumbers(
        offset_dims=(), collapsed_slice_dims=(0,), start_index_map=(0,))

    def _bcast(vec, i):
        return lax.gather(vec, jnp.full((L, 1), i, jnp.int32), bcast_dn, (1,),
                          mode=lax.GatherScatterMode.PROMISE_IN_BOUNDS)

    def idx_start(hc, k):
        base = wbase + hc * CH
        pltpu.async_copy(src_hbm.at[pl.ds(base, CH)], sb[k], si[k])
        pltpu.async_copy(dst_hbm.at[pl.ds(base, CH)], db[k], si[k])

    def idx_wait(k):
        pltpu.make_async_copy(src_hbm.at[pl.ds(0, CH)], sb[k], si[k]).wait()
        pltpu.make_async_copy(dst_hbm.at[pl.ds(0, CH)], db[k], si[k]).wait()

    def gather_start(p, k):
        pltpu.async_copy(h_hbm.at[sb[k]], rows[p], sg[p])

    def gather_wait(p, k):
        pltpu.make_async_copy(h_hbm.at[sb[k]], rows[p], sg[p]).wait()

    def scatter_start(p, k):
        pltpu.async_copy(rows[p], acc_sh.at[db[k]], ss[p], add=True)
        pltpu.async_copy(wc[p], den_sh.at[db[k]], ss[p], add=True)

    def scatter_wait(p, k):
        pltpu.make_async_copy(rows[p], acc_sh.at[db[k]], ss[p]).wait()
        pltpu.make_async_copy(wc[p], den_sh.at[db[k]], ss[p]).wait()

    pltpu.sync_copy(asad_hbm, asad_v)

    def _zrows(i, c):
        for j in range(D // L):
            rows0[i, pl.ds(j * L, L)] = zeros16
        return c
    lax.fori_loop(0, CH, _zrows, 0)

    def _zsbuf(i, c):
        sbuf[pl.ds(i * L, L)] = zeros16
        return c
    lax.fori_loop(0, RT // L, _zsbuf, 0)
    for b in range(RT // CH):
        pltpu.sync_copy(rows0, acc_sh.at[pl.ds(sid * RT + b * CH, CH)])
    pltpu.sync_copy(sbuf, den_sh.at[pl.ds(sid * RT, RT)])
    plsc.subcore_barrier()

    def compute(p, k):
        def _grp(g, cc):
            sv = sb[k][pl.ds(g * L, L)]
            dv = db[k][pl.ds(g * L, L)]
            a_s = plsc.load_gather(asad_v, [jnp.zeros((L,), jnp.int32), sv])
            a_d = plsc.load_gather(asad_v, [jnp.ones((L,), jnp.int32), dv])
            e = a_s + a_d
            e = jnp.maximum(e, e * 0.2)
            w = jnp.exp(e)
            wc[p][pl.ds(g * L, L)] = w
            for ei in range(L):
                wb = _bcast(w, ei)
                r = g * L + ei
                for j in range(D // L):
                    rows[p][r, pl.ds(j * L, L)] = \
                        rows[p][r, pl.ds(j * L, L)] * wb
            return cc
        lax.fori_loop(0, GPC, _grp, 0)

    idx_start(0, 0)
    idx_start(1, 1)
    idx_wait(0)
    gather_start(0, 0)

    def _quad(q, carry):
        for k4 in range(4):
            p = k4 & 1
            pn = p ^ 1
            hc = q * 4 + k4
            gather_wait(p, k4)

            @pl.when(hc >= 1)
            def _():
                scatter_wait(pn, (k4 + 3) % 4)

            @pl.when(hc + 1 < hcw)
            def _():
                idx_wait((k4 + 1) % 4)
                gather_start(pn, (k4 + 1) % 4)

            compute(p, k4)
            scatter_start(p, k4)

            @pl.when(hc + 2 < hcw)
            def _():
                idx_start(hc + 2, (k4 + 2) % 4)
        return carry
    lax.fori_loop(0, hcw // 4, _quad, 0)
    scatter_wait(1, 3)
    plsc.subcore_barrier()

    pltpu.sync_copy(den_sh.at[pl.ds(sid * RT, RT)], sbuf)
    for b in range(RT // CH):
        def _red(g, cc):
            dvals = sbuf[pl.ds(b * CH + g * L, L)]
            for ei in range(L):
                dsp = _bcast(dvals, ei)
                r = g * L + ei
                for j in range(D // L):
                    rows0[r, pl.ds(j * L, L)] = dsp
            return cc
        lax.fori_loop(0, GPC, _red, 0)
        pltpu.sync_copy(
            rows0, den_out.at[cid, pl.ds(sid * RT + b * CH, CH)])

    pltpu.sync_copy(acc_sh.at[pl.ds(sid * RT, RT)],
                    num_out.at[cid, pl.ds(sid * RT, RT)])


_GRID = 8
_BR = NP // _GRID


def _head_body(x_ref, W_ref, aa_ref, h_ref, asad_ref):
    h = jnp.dot(x_ref[...], W_ref[...], preferred_element_type=jnp.float32)
    h_ref[...] = h
    asad_ref[...] = lax.dot_general(
        aa_ref[...], h, (((1,), (1,)), ((), ())),
        preferred_element_type=jnp.float32)


_head = pl.pallas_call(
    _head_body,
    grid=(_GRID,),
    in_specs=[
        pl.BlockSpec((_BR, D), lambda i: (i, 0)),
        pl.BlockSpec((D, D), lambda i: (0, 0)),
        pl.BlockSpec((2, D), lambda i: (0, 0)),
    ],
    out_specs=[
        pl.BlockSpec((_BR, D), lambda i: (i, 0)),
        pl.BlockSpec((2, _BR), lambda i: (0, i)),
    ],
    out_shape=[
        jax.ShapeDtypeStruct((NP, D), jnp.float32),
        jax.ShapeDtypeStruct((2, NP), jnp.float32),
    ],
)


def _mid_body(num_ref, den_ref, b_ref, W_ref, aa_ref, h_ref, asad_ref):
    ntot = num_ref[0] + num_ref[1]
    dtot = den_ref[0] + den_ref[1]
    hact = jnp.maximum(ntot / (dtot + 1e-16) + b_ref[...], 0.0)
    h2 = jnp.dot(hact, W_ref[...], preferred_element_type=jnp.float32)
    h_ref[...] = h2
    asad_ref[...] = lax.dot_general(
        aa_ref[...], h2, (((1,), (1,)), ((), ())),
        preferred_element_type=jnp.float32)


_mid = pl.pallas_call(
    _mid_body,
    grid=(_GRID,),
    in_specs=[
        pl.BlockSpec((NC, _BR, D), lambda i: (0, i, 0)),
        pl.BlockSpec((NC, _BR, D), lambda i: (0, i, 0)),
        pl.BlockSpec((1, D), lambda i: (0, 0)),
        pl.BlockSpec((D, D), lambda i: (0, 0)),
        pl.BlockSpec((2, D), lambda i: (0, 0)),
    ],
    out_specs=[
        pl.BlockSpec((_BR, D), lambda i: (i, 0)),
        pl.BlockSpec((2, _BR), lambda i: (0, i)),
    ],
    out_shape=[
        jax.ShapeDtypeStruct((NP, D), jnp.float32),
        jax.ShapeDtypeStruct((2, NP), jnp.float32),
    ],
)


def _tail_body(num_ref, den_ref, b_ref, Wp_ref, bp_ref, batch_ref, out_ref,
               sums, cnts):
    i = pl.program_id(0)

    @pl.when(i == 0)
    def _():
        sums[...] = jnp.zeros((G, D), jnp.float32)
        cnts[...] = jnp.zeros((G, D), jnp.float32)

    ntot = num_ref[0] + num_ref[1]
    dtot = den_ref[0] + den_ref[1]
    hact = jnp.maximum(ntot / (dtot + 1e-16) + b_ref[...], 0.0)
    scores = jnp.dot(hact, Wp_ref[...],
                     preferred_element_type=jnp.float32) + bp_ref[...]
    gids = lax.broadcasted_iota(jnp.int32, (_BR, G), 1)
    oh = (batch_ref[...] == gids).astype(jnp.float32)
    sums[...] += lax.dot_general(
        oh, scores, (((0,), (0,)), ((), ())),
        preferred_element_type=jnp.float32)
    cnts[...] += lax.dot_general(
        oh, jnp.ones((_BR, D), jnp.float32), (((0,), (0,)), ((), ())),
        preferred_element_type=jnp.float32)

    @pl.when(i == _GRID - 1)
    def _():
        out_ref[...] = sums[...] / jnp.maximum(cnts[...], 1.0)


_tail = pl.pallas_call(
    _tail_body,
    grid=(_GRID,),
    in_specs=[
        pl.BlockSpec((NC, _BR, D), lambda i: (0, i, 0)),
        pl.BlockSpec((NC, _BR, D), lambda i: (0, i, 0)),
        pl.BlockSpec((1, D), lambda i: (0, 0)),
        pl.BlockSpec((D, D), lambda i: (0, 0)),
        pl.BlockSpec((1, D), lambda i: (0, 0)),
        pl.BlockSpec((_BR, 1), lambda i: (i, 0)),
    ],
    out_specs=pl.BlockSpec((G, D), lambda i: (0, 0)),
    out_shape=jax.ShapeDtypeStruct((G, D), jnp.float32),
    scratch_shapes=[
        pltpu.VMEM((G, D), jnp.float32),
        pltpu.VMEM((G, D), jnp.float32),
    ],
)


def kernel(x, edge_index, batch, W1, a1_src, a1_dst, b1, W2, a2_src, a2_dst,
           b2, Wp, bp):
    src = edge_index[0].astype(jnp.int32)
    dst = edge_index[1].astype(jnp.int32)
    src_p = jnp.concatenate([src, jnp.zeros((EP - E,), jnp.int32)])
    dst_p = jnp.concatenate([dst, jnp.full((EP - E,), DUMMY_DST, jnp.int32)])
    x_p = jnp.pad(x, ((0, NP - N), (0, 0)))
    batch_p = jnp.concatenate(
        [batch.astype(jnp.int32), jnp.full((NP - N,), G, jnp.int32)]
    ).reshape(NP, 1)
    aa1 = jnp.stack([a1_src, a1_dst])
    aa2 = jnp.stack([a2_src, a2_dst])

    h1, asad1 = _head(x_p, W1, aa1)
    num1, den1 = _gat_edges(h1, asad1, src_p, dst_p)
    h2, asad2 = _mid(num1, den1, b1.reshape(1, D), W2, aa2)
    num2, den2 = _gat_edges(h2, asad2, src_p, dst_p)
    return _tail(num2, den2, b2.reshape(1, D), Wp, bp.reshape(1, D), batch_p)
